# Initial kernel scaffold; baseline (speedup 1.0000x reference)
import jax
import jax.numpy as jnp
from jax.experimental import pallas as pl


def kernel(features, W_conv, b_conv, W_obj, b_obj, W_reg, b_reg):
    raise NotImplementedError("write your pallas kernel here")


# speedup vs baseline: 14.5105x; 14.5105x over previous
import jax
import jax.numpy as jnp
import numpy as np
from jax.experimental import pallas as pl

STRIDE = 16
ANCHOR_SIZE = 128.0
ASPECT_RATIOS = (0.2323283, 0.63365731, 1.28478321, 3.15089189)
IMG_H, IMG_W = 800, 1216
PRE_NMS_TOP_N = 6000
POST_NMS_TOP_N = 1000
NMS_THRESH = 0.7
BBOX_XFORM_CLIP = float(np.log(1000.0 / 16.0))
H_FEAT, W_FEAT = 50, 76
A = 4
NPAD = 6144
NROWS = NPAD // 128
OUT_ROWS = 1024


def _cell_anchors():
    out = []
    for r in ASPECT_RATIOS:
        w = np.round(np.sqrt(ANCHOR_SIZE * ANCHOR_SIZE / r))
        h = np.round(w * r)
        xc = yc = (STRIDE - 1.0) / 2.0
        out.append([xc - 0.5 * (w - 1), yc - 0.5 * (h - 1), xc + 0.5 * (w - 1), yc + 0.5 * (h - 1)])
    return jnp.asarray(out, dtype=jnp.float32)


def _grid_anchors(H, W):
    base = _cell_anchors()
    sx = jnp.arange(W, dtype=jnp.float32) * STRIDE
    sy = jnp.arange(H, dtype=jnp.float32) * STRIDE
    gy, gx = jnp.meshgrid(sy, sx, indexing='ij')
    shifts = jnp.stack([gx.ravel(), gy.ravel(), gx.ravel(), gy.ravel()], axis=1)
    return (shifts[:, None, :] + base[None, :, :]).reshape(-1, 4)


def _conv_same(x, w, b):
    y = jax.lax.conv_general_dilated(x, w, (1, 1), 'SAME', dimension_numbers=('NCHW', 'OIHW', 'NCHW'))
    return y + b[None, :, None, None]


def _nms_body(s_ref, dx_ref, dy_ref, dw_ref, dh_ref,
              ax1_ref, ay1_ref, ax2_ref, ay2_ref, out_ref):
    s0 = s_ref[...]
    ax1 = ax1_ref[...]
    ay1 = ay1_ref[...]
    ax2 = ax2_ref[...]
    ay2 = ay2_ref[...]
    tw = ax2 - ax1 + 1.0
    th = ay2 - ay1 + 1.0
    cx = ax1 + 0.5 * tw
    cy = ay1 + 0.5 * th
    dw = jnp.minimum(dw_ref[...], BBOX_XFORM_CLIP)
    dh = jnp.minimum(dh_ref[...], BBOX_XFORM_CLIP)
    px = dx_ref[...] * tw + cx
    py = dy_ref[...] * th + cy
    pw = jnp.exp(dw) * tw
    ph = jnp.exp(dh) * th
    x1 = jnp.clip(px - 0.5 * pw, 0.0, IMG_W - 1.0)
    y1 = jnp.clip(py - 0.5 * ph, 0.0, IMG_H - 1.0)
    x2 = jnp.clip(px + 0.5 * pw - 1.0, 0.0, IMG_W - 1.0)
    y2 = jnp.clip(py + 0.5 * ph - 1.0, 0.0, IMG_H - 1.0)
    areas = (x2 - x1 + 1.0) * (y2 - y1 + 1.0)

    iota = (jax.lax.broadcasted_iota(jnp.int32, (NROWS, 128), 0) * 128
            + jax.lax.broadcasted_iota(jnp.int32, (NROWS, 128), 1))
    lane = jax.lax.broadcasted_iota(jnp.int32, (1, 128), 1)
    neg_inf = jnp.float32(-jnp.inf)

    def body(t, sw):
        m = jnp.max(sw)
        idx = jnp.min(jnp.where(sw == m, iota, jnp.int32(NPAD)))
        sel = iota == idx
        bx1 = jnp.max(jnp.where(sel, x1, neg_inf))
        by1 = jnp.max(jnp.where(sel, y1, neg_inf))
        bx2 = jnp.max(jnp.where(sel, x2, neg_inf))
        by2 = jnp.max(jnp.where(sel, y2, neg_inf))
        barea = jnp.max(jnp.where(sel, areas, neg_inf))
        bscore = jnp.max(jnp.where(sel, s0, neg_inf))
        xx1 = jnp.maximum(bx1, x1)
        yy1 = jnp.maximum(by1, y1)
        xx2 = jnp.minimum(bx2, x2)
        yy2 = jnp.minimum(by2, y2)
        inter = jnp.maximum(0.0, xx2 - xx1 + 1.0) * jnp.maximum(0.0, yy2 - yy1 + 1.0)
        iou = inter / (barea + areas - inter)
        row = jnp.where(lane == 0, bx1,
              jnp.where(lane == 1, by1,
              jnp.where(lane == 2, bx2,
              jnp.where(lane == 3, by2, bscore))))
        out_ref[pl.ds(t, 1), :] = row
        return jnp.where(iou > NMS_THRESH, neg_inf, sw)

    jax.lax.fori_loop(0, POST_NMS_TOP_N, body, s0)


def kernel(features, W_conv, b_conv, W_obj, b_obj, W_reg, b_reg):
    t = jax.nn.relu(_conv_same(features, W_conv, b_conv))
    obj = _conv_same(t, W_obj, b_obj)
    reg = _conv_same(t, W_reg, b_reg)
    B, _, H, W = obj.shape
    obj = jnp.transpose(obj, (0, 2, 3, 1)).reshape(B, -1)
    reg = reg.reshape(B, A, 4, H, W)
    reg = jnp.transpose(reg, (0, 3, 4, 1, 2)).reshape(B, -1, 4)
    anchors = _grid_anchors(H, W)
    scores = jax.nn.sigmoid(obj[0])
    top_scores, top_idx = jax.lax.top_k(scores, PRE_NMS_TOP_N)
    codes = reg[0][top_idx]
    anc = anchors[top_idx]

    def pad(v, fill):
        return jnp.full((NPAD,), fill, jnp.float32).at[:PRE_NMS_TOP_N].set(v).reshape(NROWS, 128)

    args = (
        pad(top_scores, -jnp.inf),
        pad(codes[:, 0], 0.0), pad(codes[:, 1], 0.0),
        pad(codes[:, 2], 0.0), pad(codes[:, 3], 0.0),
        pad(anc[:, 0], 0.0), pad(anc[:, 1], 0.0),
        pad(anc[:, 2], 15.0), pad(anc[:, 3], 15.0),
    )
    out = pl.pallas_call(
        _nms_body,
        out_shape=jax.ShapeDtypeStruct((OUT_ROWS, 128), jnp.float32),
    )(*args)
    return out[:POST_NMS_TOP_N, :5]


# speedup vs baseline: 17.6570x; 1.2168x over previous
import jax
import jax.numpy as jnp
import numpy as np
from jax.experimental import pallas as pl
from jax.experimental.pallas import tpu as pltpu

STRIDE = 16
ANCHOR_SIZE = 128.0
ASPECT_RATIOS = (0.2323283, 0.63365731, 1.28478321, 3.15089189)
IMG_H, IMG_W = 800, 1216
PRE_NMS_TOP_N = 6000
POST_NMS_TOP_N = 1000
NMS_THRESH = 0.7
BBOX_XFORM_CLIP = float(np.log(1000.0 / 16.0))
H_FEAT, W_FEAT = 50, 76
A = 4
NPAD = 6144
NROWS = NPAD // 128
OUT_ROWS = 1024


def _cell_anchors():
    out = []
    for r in ASPECT_RATIOS:
        w = np.round(np.sqrt(ANCHOR_SIZE * ANCHOR_SIZE / r))
        h = np.round(w * r)
        xc = yc = (STRIDE - 1.0) / 2.0
        out.append([xc - 0.5 * (w - 1), yc - 0.5 * (h - 1), xc + 0.5 * (w - 1), yc + 0.5 * (h - 1)])
    return jnp.asarray(out, dtype=jnp.float32)


def _grid_anchors(H, W):
    base = _cell_anchors()
    sx = jnp.arange(W, dtype=jnp.float32) * STRIDE
    sy = jnp.arange(H, dtype=jnp.float32) * STRIDE
    gy, gx = jnp.meshgrid(sy, sx, indexing='ij')
    shifts = jnp.stack([gx.ravel(), gy.ravel(), gx.ravel(), gy.ravel()], axis=1)
    return (shifts[:, None, :] + base[None, :, :]).reshape(-1, 4)


def _conv_same(x, w, b):
    y = jax.lax.conv_general_dilated(x, w, (1, 1), 'SAME', dimension_numbers=('NCHW', 'OIHW', 'NCHW'))
    return y + b[None, :, None, None]


def _nms_body(s_ref, dx_ref, dy_ref, dw_ref, dh_ref,
              ax1_ref, ay1_ref, ax2_ref, ay2_ref, out_ref,
              sw_ref, x1_ref, y1_ref, x2_ref, y2_ref, ar_ref, sc_ref):
    s0 = s_ref[...]
    ax1 = ax1_ref[...]
    ay1 = ay1_ref[...]
    ax2 = ax2_ref[...]
    ay2 = ay2_ref[...]
    tw = ax2 - ax1 + 1.0
    th = ay2 - ay1 + 1.0
    cx = ax1 + 0.5 * tw
    cy = ay1 + 0.5 * th
    dw = jnp.minimum(dw_ref[...], BBOX_XFORM_CLIP)
    dh = jnp.minimum(dh_ref[...], BBOX_XFORM_CLIP)
    px = dx_ref[...] * tw + cx
    py = dy_ref[...] * th + cy
    pw = jnp.exp(dw) * tw
    ph = jnp.exp(dh) * th
    x1 = jnp.clip(px - 0.5 * pw, 0.0, IMG_W - 1.0)
    y1 = jnp.clip(py - 0.5 * ph, 0.0, IMG_H - 1.0)
    x2 = jnp.clip(px + 0.5 * pw - 1.0, 0.0, IMG_W - 1.0)
    y2 = jnp.clip(py + 0.5 * ph - 1.0, 0.0, IMG_H - 1.0)
    areas = (x2 - x1 + 1.0) * (y2 - y1 + 1.0)
    sw_ref[...] = s0
    x1_ref[...] = x1
    y1_ref[...] = y1
    x2_ref[...] = x2
    y2_ref[...] = y2
    ar_ref[...] = areas
    sc_ref[...] = s0

    iota = (jax.lax.broadcasted_iota(jnp.int32, (NROWS, 128), 0) * 128
            + jax.lax.broadcasted_iota(jnp.int32, (NROWS, 128), 1))
    lane = jax.lax.broadcasted_iota(jnp.int32, (1, 128), 1)
    neg_inf = jnp.float32(-jnp.inf)

    def extract(ref, r, lsel):
        return jnp.max(jnp.where(lsel, ref[pl.ds(r, 1), :], neg_inf))

    def body(t, carry):
        sw = sw_ref[...]
        first = jnp.min(jnp.where(sw > neg_inf, iota, jnp.int32(NPAD)))
        idx = jnp.where(first == jnp.int32(NPAD), jnp.int32(0), first)
        r = idx // 128
        lsel = lane == (idx % 128)
        bx1 = extract(x1_ref, r, lsel)
        by1 = extract(y1_ref, r, lsel)
        bx2 = extract(x2_ref, r, lsel)
        by2 = extract(y2_ref, r, lsel)
        barea = extract(ar_ref, r, lsel)
        bscore = extract(sc_ref, r, lsel)
        xx1 = jnp.maximum(bx1, x1)
        yy1 = jnp.maximum(by1, y1)
        xx2 = jnp.minimum(bx2, x2)
        yy2 = jnp.minimum(by2, y2)
        inter = jnp.maximum(0.0, xx2 - xx1 + 1.0) * jnp.maximum(0.0, yy2 - yy1 + 1.0)
        iou = inter / (barea + areas - inter)
        row = jnp.where(lane == 0, bx1,
              jnp.where(lane == 1, by1,
              jnp.where(lane == 2, bx2,
              jnp.where(lane == 3, by2, bscore))))
        out_ref[pl.ds(t, 1), :] = row
        sw_ref[...] = jnp.where(iou > NMS_THRESH, neg_inf, sw)
        return carry

    jax.lax.fori_loop(0, POST_NMS_TOP_N, body, 0)


def kernel(features, W_conv, b_conv, W_obj, b_obj, W_reg, b_reg):
    t = jax.nn.relu(_conv_same(features, W_conv, b_conv))
    obj = _conv_same(t, W_obj, b_obj)
    reg = _conv_same(t, W_reg, b_reg)
    B, _, H, W = obj.shape
    obj = jnp.transpose(obj, (0, 2, 3, 1)).reshape(B, -1)
    reg = reg.reshape(B, A, 4, H, W)
    reg = jnp.transpose(reg, (0, 3, 4, 1, 2)).reshape(B, -1, 4)
    anchors = _grid_anchors(H, W)
    scores = jax.nn.sigmoid(obj[0])
    top_scores, top_idx = jax.lax.top_k(scores, PRE_NMS_TOP_N)
    codes = reg[0][top_idx]
    anc = anchors[top_idx]

    def pad(v, fill):
        return jnp.full((NPAD,), fill, jnp.float32).at[:PRE_NMS_TOP_N].set(v).reshape(NROWS, 128)

    args = (
        pad(top_scores, -jnp.inf),
        pad(codes[:, 0], 0.0), pad(codes[:, 1], 0.0),
        pad(codes[:, 2], 0.0), pad(codes[:, 3], 0.0),
        pad(anc[:, 0], 0.0), pad(anc[:, 1], 0.0),
        pad(anc[:, 2], 15.0), pad(anc[:, 3], 15.0),
    )
    out = pl.pallas_call(
        _nms_body,
        out_shape=jax.ShapeDtypeStruct((OUT_ROWS, 128), jnp.float32),
        scratch_shapes=[pltpu.VMEM((NROWS, 128), jnp.float32)] * 7,
    )(*args)
    return out[:POST_NMS_TOP_N, :5]
